# baseline (speedup 1.0000x reference)
import jax
import jax.numpy as jnp
from jax import lax
from jax.experimental import pallas as pl
from jax.experimental.pallas import tpu as pltpu
from jax.experimental.pallas import tpu_sc as plsc

VOCAB = 100000
D = 128
B = 1024
L = 512
N = B * L

NSLICE = 4
M = N // NSLICE

NC = 2
NS = 16
NW = NC * NS

ROWS_PER_W = M // NW
CHUNK = 128
NBUF = 4
NCHUNK = ROWS_PER_W // CHUNK
NOUTER = NCHUNK // NBUF

RB = 2048
SEQ_PER_RB = RB // L
NBLK = M // RB


def _sc_gather_body(x_hbm, table_hbm, out_hbm, idx_v, rows_v,
                    isem, gsem, osem):
  wid = lax.axis_index("s") * NC + lax.axis_index("c")
  base_w = wid * ROWS_PER_W

  def idx_copy(b, c):
    src = x_hbm.at[pl.ds(base_w + c * CHUNK, CHUNK)]
    return pltpu.make_async_copy(src, idx_v.at[b], isem.at[b])

  def gather_copy(b):
    return pltpu.make_async_copy(table_hbm.at[idx_v.at[b]], rows_v.at[b],
                                 gsem.at[b])

  def out_copy(b, c):
    dst = out_hbm.at[pl.ds(base_w + c * CHUNK, CHUNK)]
    return pltpu.make_async_copy(rows_v.at[b], dst, osem.at[b])

  for b in range(NBUF):
    idx_copy(b, b).start()

  def outer(k, _):
    for b in range(NBUF):
      c = k * NBUF + b

      @pl.when(k > 0)
      def _wait_prev_out():
        out_copy(b, c - NBUF).wait()

      idx_copy(b, c).wait()
      gather_copy(b).start()

    for b in range(NBUF):
      c = k * NBUF + b
      gather_copy(b).wait()
      out_copy(b, c).start()

      @pl.when(k < NOUTER - 1)
      def _prefetch_idx():
        idx_copy(b, c + NBUF).start()

    return _

  lax.fori_loop(0, NOUTER, outer, None)

  for b in range(NBUF):
    out_copy(b, (NOUTER - 1) * NBUF + b).wait()


def _sc_gather(xf, table):
  mesh = plsc.VectorSubcoreMesh(core_axis_name="c", subcore_axis_name="s")
  return pl.kernel(
      _sc_gather_body,
      out_type=jax.ShapeDtypeStruct((M, D), jnp.float32),
      mesh=mesh,
      scratch_types=[
          pltpu.VMEM((NBUF, CHUNK), jnp.int32),
          pltpu.VMEM((NBUF, CHUNK, D), jnp.float32),
          pltpu.SemaphoreType.DMA((NBUF,)),
          pltpu.SemaphoreType.DMA((NBUF,)),
          pltpu.SemaphoreType.DMA((NBUF,)),
      ],
  )(xf, table)


def _ln_body(tok_ref, seg_ref, pe_ref, segtab_ref, gamma_ref, beta_ref,
             carry_ref, o_ref):
  del carry_ref
  s = seg_ref[0, 0, :][:, None]
  oh = (s == lax.broadcasted_iota(jnp.int32, (1, 3), 1)).astype(jnp.float32)
  seg_emb = jnp.dot(oh, segtab_ref[...], preferred_element_type=jnp.float32)
  h = tok_ref[...] + seg_emb
  h = (h.reshape(SEQ_PER_RB, L, D) + pe_ref[...][None]).reshape(RB, D)
  ones = jnp.full((D, D), 1.0 / D, jnp.float32)
  mean = jnp.dot(h, ones, preferred_element_type=jnp.float32)
  e2 = jnp.dot(h * h, ones, preferred_element_type=jnp.float32)
  var = e2 - mean * mean
  inv = lax.rsqrt(var + 1e-5)
  scale = inv * gamma_ref[...]
  o_ref[...] = h * scale + (beta_ref[...] - mean * scale)


def _tc_ln_slice(j, tok_rows, seg3d_j, pe2d, seg_table, gamma, beta, carry):
  return pl.pallas_call(
      _ln_body,
      grid=(NBLK,),
      in_specs=[
          pl.BlockSpec((RB, D), lambda i: (i, 0)),
          pl.BlockSpec((1, 1, RB), lambda i: (i, 0, 0)),
          pl.BlockSpec((L, D), lambda i: (0, 0)),
          pl.BlockSpec((3, D), lambda i: (0, 0)),
          pl.BlockSpec((1, D), lambda i: (0, 0)),
          pl.BlockSpec((1, D), lambda i: (0, 0)),
          pl.BlockSpec(memory_space=pl.ANY),
      ],
      out_specs=pl.BlockSpec((RB, D), lambda i, j=j: (i + j * NBLK, 0)),
      out_shape=jax.ShapeDtypeStruct((N, D), jnp.float32),
      input_output_aliases={6: 0},
  )(tok_rows, seg3d_j, pe2d, seg_table, gamma, beta, carry)


@jax.jit
def _pipeline(xf, seg3d, table, seg_table, gamma, beta, pe2d):
  out = jnp.zeros((N, D), jnp.float32)
  for j in range(NSLICE):
    rows_j = _sc_gather(lax.dynamic_slice_in_dim(xf, j * M, M), table)
    out = _tc_ln_slice(j, rows_j,
                       lax.dynamic_slice_in_dim(seg3d, j * NBLK, NBLK),
                       pe2d, seg_table, gamma, beta, out)
  return out


def kernel(x, seg, tok_table, seg_table, gamma, beta, pe):
  xf = x.reshape(-1)
  seg3d = seg.reshape(-1, 1, RB)
  pe2d = pe.reshape(pe.shape[1], D)[:L]
  out = _pipeline(xf, seg3d, tok_table, seg_table,
                  gamma.reshape(1, D), beta.reshape(1, D), pe2d)
  return out.reshape(B, L, D)


# speedup vs baseline: 1.2842x; 1.2842x over previous
import jax
import jax.numpy as jnp
from jax import lax
from jax.experimental import pallas as pl
from jax.experimental.pallas import tpu as pltpu
from jax.experimental.pallas import tpu_sc as plsc

VOCAB = 100000
D = 128
B = 1024
L = 512
N = B * L

NSLICE = 4
M = N // NSLICE

NC = 2
NS = 16
NW = NC * NS

ROWS_PER_W = M // NW
CHUNK = 128
NBUF = 4
NCHUNK = ROWS_PER_W // CHUNK
NOUTER = NCHUNK // NBUF

RB = 2048
SEQ_PER_RB = RB // L
NBLK = M // RB


def _sc_gather_body(x_hbm, table_hbm, out_hbm, idx_v, rows_v,
                    isem, gsem, osem):
  wid = lax.axis_index("s") * NC + lax.axis_index("c")
  base_w = wid * ROWS_PER_W

  def idx_copy(b, c):
    src = x_hbm.at[pl.ds(base_w + c * CHUNK, CHUNK)]
    return pltpu.make_async_copy(src, idx_v.at[b], isem.at[b])

  def gather_copy(b):
    return pltpu.make_async_copy(table_hbm.at[idx_v.at[b]], rows_v.at[b],
                                 gsem.at[b])

  def out_copy(b, c):
    dst = out_hbm.at[pl.ds(base_w + c * CHUNK, CHUNK)]
    return pltpu.make_async_copy(rows_v.at[b], dst, osem.at[b])

  for b in range(NBUF):
    idx_copy(b, b).start()

  def outer(k, _):
    for b in range(NBUF):
      c = k * NBUF + b

      @pl.when(k > 0)
      def _wait_prev_out():
        out_copy(b, c - NBUF).wait()

      idx_copy(b, c).wait()
      gather_copy(b).start()

    for b in range(NBUF):
      c = k * NBUF + b
      gather_copy(b).wait()
      out_copy(b, c).start()

      @pl.when(k < NOUTER - 1)
      def _prefetch_idx():
        idx_copy(b, c + NBUF).start()

    return _

  lax.fori_loop(0, NOUTER, outer, None)

  for b in range(NBUF):
    out_copy(b, (NOUTER - 1) * NBUF + b).wait()


def _sc_gather(xf, table):
  mesh = plsc.VectorSubcoreMesh(core_axis_name="c", subcore_axis_name="s")
  return pl.kernel(
      _sc_gather_body,
      out_type=jax.ShapeDtypeStruct((M, D), jnp.float32),
      mesh=mesh,
      scratch_types=[
          pltpu.VMEM((NBUF, CHUNK), jnp.int32),
          pltpu.VMEM((NBUF, CHUNK, D), jnp.float32),
          pltpu.SemaphoreType.DMA((NBUF,)),
          pltpu.SemaphoreType.DMA((NBUF,)),
          pltpu.SemaphoreType.DMA((NBUF,)),
      ],
  )(xf, table)


def _ln_body_nocarry(tok_ref, seg_ref, pe_ref, segtab_ref, gamma_ref,
                     beta_ref, o_ref):
  _ln_impl(tok_ref, seg_ref, pe_ref, segtab_ref, gamma_ref, beta_ref, o_ref)


def _ln_body(tok_ref, seg_ref, pe_ref, segtab_ref, gamma_ref, beta_ref,
             carry_ref, o_ref):
  del carry_ref
  _ln_impl(tok_ref, seg_ref, pe_ref, segtab_ref, gamma_ref, beta_ref, o_ref)


def _ln_impl(tok_ref, seg_ref, pe_ref, segtab_ref, gamma_ref, beta_ref,
             o_ref):
  s = seg_ref[0, 0, :][:, None]
  oh = (s == lax.broadcasted_iota(jnp.int32, (1, 3), 1)).astype(jnp.float32)
  seg_emb = jnp.dot(oh, segtab_ref[...], preferred_element_type=jnp.float32)
  h = tok_ref[...] + seg_emb
  h = (h.reshape(SEQ_PER_RB, L, D) + pe_ref[...][None]).reshape(RB, D)
  ones = jnp.full((D, D), 1.0 / D, jnp.float32)
  mean = jnp.dot(h, ones, preferred_element_type=jnp.float32)
  e2 = jnp.dot(h * h, ones, preferred_element_type=jnp.float32)
  var = e2 - mean * mean
  inv = lax.rsqrt(var + 1e-5)
  scale = inv * gamma_ref[...]
  o_ref[...] = h * scale + (beta_ref[...] - mean * scale)


def _tc_ln_slice(j, tok_rows, seg3d_j, pe2d, seg_table, gamma, beta, carry):
  in_specs = [
      pl.BlockSpec((RB, D), lambda i: (i, 0)),
      pl.BlockSpec((1, 1, RB), lambda i: (i, 0, 0)),
      pl.BlockSpec((L, D), lambda i: (0, 0)),
      pl.BlockSpec((3, D), lambda i: (0, 0)),
      pl.BlockSpec((1, D), lambda i: (0, 0)),
      pl.BlockSpec((1, D), lambda i: (0, 0)),
  ]
  args = [tok_rows, seg3d_j, pe2d, seg_table, gamma, beta]
  aliases = {}
  if carry is not None:
    in_specs.append(pl.BlockSpec(memory_space=pl.ANY))
    args.append(carry)
    aliases = {6: 0}
  body = _ln_body if carry is not None else _ln_body_nocarry
  return pl.pallas_call(
      body,
      grid=(NBLK,),
      in_specs=in_specs,
      out_specs=pl.BlockSpec((RB, D), lambda i, j=j: (i + j * NBLK, 0)),
      out_shape=jax.ShapeDtypeStruct((N, D), jnp.float32),
      input_output_aliases=aliases,
  )(*args)


@jax.jit
def _pipeline(xf, seg3d, table, seg_table, gamma, beta, pe2d):
  out = None
  for j in range(NSLICE):
    rows_j = _sc_gather(lax.dynamic_slice_in_dim(xf, j * M, M), table)
    out = _tc_ln_slice(j, rows_j,
                       lax.dynamic_slice_in_dim(seg3d, j * NBLK, NBLK),
                       pe2d, seg_table, gamma, beta, out)
  return out


def kernel(x, seg, tok_table, seg_table, gamma, beta, pe):
  xf = x.reshape(-1)
  seg3d = seg.reshape(-1, 1, RB)
  pe2d = pe.reshape(pe.shape[1], D)[:L]
  out = _pipeline(xf, seg3d, tok_table, seg_table,
                  gamma.reshape(1, D), beta.reshape(1, D), pe2d)
  return out.reshape(B, L, D)


# speedup vs baseline: 1.2889x; 1.0037x over previous
import jax
import jax.numpy as jnp
from jax import lax
from jax.experimental import pallas as pl
from jax.experimental.pallas import tpu as pltpu
from jax.experimental.pallas import tpu_sc as plsc

VOCAB = 100000
D = 128
B = 1024
L = 512
N = B * L

NSLICE = 8
M = N // NSLICE

NC = 2
NS = 16
NW = NC * NS

ROWS_PER_W = M // NW
CHUNK = 128
NBUF = 4
NCHUNK = ROWS_PER_W // CHUNK
NOUTER = NCHUNK // NBUF

RB = 2048
SEQ_PER_RB = RB // L
NBLK = M // RB


def _sc_gather_body(x_hbm, table_hbm, out_hbm, idx_v, rows_v,
                    isem, gsem, osem):
  wid = lax.axis_index("s") * NC + lax.axis_index("c")
  base_w = wid * ROWS_PER_W

  def idx_copy(b, c):
    src = x_hbm.at[pl.ds(base_w + c * CHUNK, CHUNK)]
    return pltpu.make_async_copy(src, idx_v.at[b], isem.at[b])

  def gather_copy(b):
    return pltpu.make_async_copy(table_hbm.at[idx_v.at[b]], rows_v.at[b],
                                 gsem.at[b])

  def out_copy(b, c):
    dst = out_hbm.at[pl.ds(base_w + c * CHUNK, CHUNK)]
    return pltpu.make_async_copy(rows_v.at[b], dst, osem.at[b])

  for b in range(NBUF):
    idx_copy(b, b).start()

  def outer(k, _):
    for b in range(NBUF):
      c = k * NBUF + b

      @pl.when(k > 0)
      def _wait_prev_out():
        out_copy(b, c - NBUF).wait()

      idx_copy(b, c).wait()
      gather_copy(b).start()

    for b in range(NBUF):
      c = k * NBUF + b
      gather_copy(b).wait()
      out_copy(b, c).start()

      @pl.when(k < NOUTER - 1)
      def _prefetch_idx():
        idx_copy(b, c + NBUF).start()

    return _

  lax.fori_loop(0, NOUTER, outer, None)

  for b in range(NBUF):
    out_copy(b, (NOUTER - 1) * NBUF + b).wait()


def _sc_gather(xf, table):
  mesh = plsc.VectorSubcoreMesh(core_axis_name="c", subcore_axis_name="s")
  return pl.kernel(
      _sc_gather_body,
      out_type=jax.ShapeDtypeStruct((M, D), jnp.float32),
      mesh=mesh,
      scratch_types=[
          pltpu.VMEM((NBUF, CHUNK), jnp.int32),
          pltpu.VMEM((NBUF, CHUNK, D), jnp.float32),
          pltpu.SemaphoreType.DMA((NBUF,)),
          pltpu.SemaphoreType.DMA((NBUF,)),
          pltpu.SemaphoreType.DMA((NBUF,)),
      ],
  )(xf, table)


def _ln_body_nocarry(tok_ref, seg_ref, pe_ref, segtab_ref, gamma_ref,
                     beta_ref, o_ref):
  _ln_impl(tok_ref, seg_ref, pe_ref, segtab_ref, gamma_ref, beta_ref, o_ref)


def _ln_body(tok_ref, seg_ref, pe_ref, segtab_ref, gamma_ref, beta_ref,
             carry_ref, o_ref):
  del carry_ref
  _ln_impl(tok_ref, seg_ref, pe_ref, segtab_ref, gamma_ref, beta_ref, o_ref)


def _ln_impl(tok_ref, seg_ref, pe_ref, segtab_ref, gamma_ref, beta_ref,
             o_ref):
  s = seg_ref[0, 0, :][:, None]
  oh = (s == lax.broadcasted_iota(jnp.int32, (1, 3), 1)).astype(jnp.float32)
  seg_emb = jnp.dot(oh, segtab_ref[...], preferred_element_type=jnp.float32)
  h = tok_ref[...] + seg_emb
  h = (h.reshape(SEQ_PER_RB, L, D) + pe_ref[...][None]).reshape(RB, D)
  ones = jnp.full((D, D), 1.0 / D, jnp.float32)
  mean = jnp.dot(h, ones, preferred_element_type=jnp.float32)
  e2 = jnp.dot(h * h, ones, preferred_element_type=jnp.float32)
  var = e2 - mean * mean
  inv = lax.rsqrt(var + 1e-5)
  scale = inv * gamma_ref[...]
  o_ref[...] = h * scale + (beta_ref[...] - mean * scale)


def _tc_ln_slice(j, tok_rows, seg3d_j, pe2d, seg_table, gamma, beta, carry):
  in_specs = [
      pl.BlockSpec((RB, D), lambda i: (i, 0)),
      pl.BlockSpec((1, 1, RB), lambda i: (i, 0, 0)),
      pl.BlockSpec((L, D), lambda i: (0, 0)),
      pl.BlockSpec((3, D), lambda i: (0, 0)),
      pl.BlockSpec((1, D), lambda i: (0, 0)),
      pl.BlockSpec((1, D), lambda i: (0, 0)),
  ]
  args = [tok_rows, seg3d_j, pe2d, seg_table, gamma, beta]
  aliases = {}
  if carry is not None:
    in_specs.append(pl.BlockSpec(memory_space=pl.ANY))
    args.append(carry)
    aliases = {6: 0}
  body = _ln_body if carry is not None else _ln_body_nocarry
  return pl.pallas_call(
      body,
      grid=(NBLK,),
      in_specs=in_specs,
      out_specs=pl.BlockSpec((RB, D), lambda i, j=j: (i + j * NBLK, 0)),
      out_shape=jax.ShapeDtypeStruct((N, D), jnp.float32),
      input_output_aliases=aliases,
  )(*args)


@jax.jit
def _pipeline(xf, seg3d, table, seg_table, gamma, beta, pe2d):
  out = None
  for j in range(NSLICE):
    rows_j = _sc_gather(lax.dynamic_slice_in_dim(xf, j * M, M), table)
    out = _tc_ln_slice(j, rows_j,
                       lax.dynamic_slice_in_dim(seg3d, j * NBLK, NBLK),
                       pe2d, seg_table, gamma, beta, out)
  return out


def kernel(x, seg, tok_table, seg_table, gamma, beta, pe):
  xf = x.reshape(-1)
  seg3d = seg.reshape(-1, 1, RB)
  pe2d = pe.reshape(pe.shape[1], D)[:L]
  out = _pipeline(xf, seg3d, tok_table, seg_table,
                  gamma.reshape(1, D), beta.reshape(1, D), pe2d)
  return out.reshape(B, L, D)
